# Initial kernel scaffold; baseline (speedup 1.0000x reference)
import jax
import jax.numpy as jnp
from jax.experimental import pallas as pl


def kernel(x, edge_index, node_type_index, edge_type_index, l1_node_w, l1_edge_w, l1_node_b, l2_node_w, l2_edge_w, l2_node_b, l3_node_w, l3_edge_w, l3_node_b, l4_node_w, l4_edge_w, l4_node_b, spatial_Wl, spatial_bl, spatial_Wr, spatial_br, spatial_W1, spatial_b1, spatial_W2, spatial_b2, reduce_Wl, reduce_bl, reduce_Wr, reduce_br, reduce_W1, reduce_b1, reduce_W2, reduce_b2, fuse_Wl, fuse_bl, fuse_Wr, fuse_br, fuse_W1, fuse_b1, fuse_W2, fuse_b2, reorder_Wl, reorder_bl, reorder_Wr, reorder_br, reorder_W1, reorder_b1, reorder_W2, reorder_b2, unroll_Wl, unroll_bl, unroll_Wr, unroll_br, unroll_W1, unroll_b1, unroll_W2, unroll_b2, spatial_choices, reduce_choices, fuse_choices, reorder_choices, unroll_choices):
    raise NotImplementedError("write your pallas kernel here")


# speedup vs baseline: 15.0509x; 15.0509x over previous
import functools

import jax
import jax.numpy as jnp
from jax import lax
from jax.experimental import pallas as pl
from jax.experimental.pallas import tpu as pltpu
from jax.experimental.pallas import tpu_sc as plsc

_N = 10000
_E = 320000
_NT = 4
_ET = 5
_NTI = [0, 4000, 6000, 8000, 10000]
_BN = 1000
_NB = _N // _BN
_CH = 80
_EPT = _E // _ET
_TPS = _EPT // 16
_NCH = _TPS // _CH
_RPT = _N // 16


def _blk_type(b):
    b = jnp.asarray(b)
    return ((b >= 4).astype(jnp.int32) + (b >= 6).astype(jnp.int32)
            + (b >= 8).astype(jnp.int32))


def _dense_stage(a_prev, nw, wat, f, half, nq, acc_w, prev=None,
                 emit_o=False):
    ic = nw.shape[1]
    npz = 0 if prev is None else len(prev[0])

    def body(*refs):
        if prev is None:
            a_ref, nw_ref, wat_ref = refs[:3]
            outs = refs[3:]
            a = a_ref[...]
        else:
            part_refs = refs[:npz]
            denT_ref, bias_ref, nw_ref, wat_ref = refs[npz:npz + 4]
            outs = refs[npz + 4:]
            hp, ocp = prev[3], prev[4]
            agg = jnp.concatenate([r[...] for r in part_refs], axis=1)
            d = denT_ref[...][:, :hp]
            o = agg.reshape(_BN, hp, ocp) / (d[:, :, None] + 1e-16)
            o = o.reshape(_BN, hp * ocp) + bias_ref[0]
            if emit_o:
                outs[2][...] = o
            a = jnp.maximum(o, 0.0)
        lin = jnp.maximum(jnp.dot(a, nw_ref[0],
                                  preferred_element_type=jnp.float32), 0.0)
        at = jnp.dot(lin, wat_ref[...], preferred_element_type=jnp.float32)
        for qq in range(nq):
            outs[0][qq] = lin[:, qq * acc_w:(qq + 1) * acc_w]
        outs[1][...] = at

    out_shapes = [jax.ShapeDtypeStruct((nq, _N, acc_w), jnp.float32),
                  jax.ShapeDtypeStruct((_N, 32), jnp.float32)]
    out_specs = [pl.BlockSpec((nq, _BN, acc_w), lambda b: (0, b, 0)),
                 pl.BlockSpec((_BN, 32), lambda b: (b, 0))]
    if prev is None:
        in_specs = [pl.BlockSpec((_BN, ic), lambda b: (b, 0)),
                    pl.BlockSpec((1, ic, f), lambda b: (_blk_type(b), 0, 0)),
                    pl.BlockSpec((f, 32), lambda b: (0, 0))]
        args = (a_prev, nw, wat)
    else:
        parts, den, bias, hp, ocp = prev
        fp = hp * ocp
        if emit_o:
            out_shapes.append(jax.ShapeDtypeStruct((_N, fp), jnp.float32))
            out_specs.append(pl.BlockSpec((_BN, fp), lambda b: (b, 0)))
        aw = parts[0].shape[1]
        in_specs = ([pl.BlockSpec((_BN, aw), lambda b: (b, 0))
                     for _ in range(npz)]
                    + [pl.BlockSpec((_BN, 8), lambda b: (b, 0)),
                       pl.BlockSpec((1, 1, fp),
                                    lambda b: (_blk_type(b), 0, 0)),
                       pl.BlockSpec((1, ic, f),
                                    lambda b: (_blk_type(b), 0, 0)),
                       pl.BlockSpec((f, 32), lambda b: (0, 0))])
        args = tuple(parts) + (den, bias.reshape(_NT, 1, fp), nw, wat)

    return pl.pallas_call(
        body, grid=(_NB,), in_specs=in_specs, out_specs=out_specs,
        out_shape=out_shapes)(*args)


def _transpose_stage(atp):
    def body(a_ref, o_ref):
        o_ref[...] = a_ref[...].T

    return pl.pallas_call(
        body, grid=(8,),
        in_specs=[pl.BlockSpec((1280, 32), lambda b: (b, 0))],
        out_specs=pl.BlockSpec((32, 1280), lambda b: (0, b)),
        out_shape=jax.ShapeDtypeStruct((32, 10240), jnp.float32))(atp)


def _ub_stage(atT, h):
    def body(at_ref, out_ref, mx_ref):
        p = pl.program_id(0)
        b = pl.program_id(1)

        @pl.when(jnp.logical_and(p == 0, b == 0))
        def _():
            mx_ref[...] = jnp.full((32, 1280), -jnp.inf, jnp.float32)

        @pl.when(p == 0)
        def _():
            mx_ref[...] = jnp.maximum(mx_ref[...], at_ref[...])

        maj = jnp.max(mx_ref[...], axis=1)
        blk = at_ref[...]
        rows = []
        for bb in range(h):
            u = jnp.full((1280,), -jnp.inf, jnp.float32)
            for t in range(_ET):
                u = jnp.maximum(u, blk[t * 2 * h + bb, :]
                                + maj[t * 2 * h + h + bb])
            rows.append(jnp.where(u >= 0, u, 0.2 * u))
        for bb in range(8 - h):
            rows.append(jnp.zeros((1280,), jnp.float32))
        out_ref[...] = jnp.stack(rows, axis=0)

    return pl.pallas_call(
        body, grid=(2, 8),
        in_specs=[pl.BlockSpec((32, 1280), lambda p, b: (0, b))],
        out_specs=pl.BlockSpec((8, 1280), lambda p, b: (0, b)),
        out_shape=jax.ShapeDtypeStruct((8, 10240), jnp.float32),
        scratch_shapes=[pltpu.VMEM((32, 1280), jnp.float32)])(atT)


def _sc_edge_stage(lin_stk, atT, ubT, src, dst, half, h):
    acc_w = half if h == 1 else 64
    npass = half // acc_w
    nq = 2 * npass
    mesh = plsc.VectorSubcoreMesh(core_axis_name="c", subcore_axis_name="s",
                                  num_cores=2)

    def body(lin_hbm, atT_hbm, ubT_hbm, src_hbm, dst_hbm,
             agg_hbm, den_hbm,
             acc_sp, den_sp, ai_v, aj_v, ub_v, zb_v, zd_v,
             srcb, dstb, sadj, dind, exb, rows_v, sem):
        cid = lax.axis_index("c")
        sid = lax.axis_index("s")

        def zb_body(i, zc):
            for j in range(acc_w // 16):
                zb_v[i, pl.ds(j * 16, 16)] = jnp.zeros((16,), jnp.float32)
            return zc
        lax.fori_loop(0, 125, zb_body, 0)

        def zd_body(i, zc):
            zd_v[pl.ds(i * 16, 16)] = jnp.zeros((16,), jnp.float32)
            return zc
        lax.fori_loop(0, (_RPT * 8 + 15) // 16, zd_body, 0)

        for bb in range(h):
            pltpu.sync_copy(ubT_hbm.at[pl.ds(bb * 10240, _N)],
                            ub_v.at[pl.ds(bb * 10240, _N)])

        for p in range(npass):
            for k in range(5):
                pltpu.sync_copy(zb_v,
                                acc_sp.at[pl.ds(sid * _RPT + k * 125, 125)])
            if p == 0:
                pltpu.sync_copy(zd_v.at[pl.ds(0, _RPT * 8)],
                                den_sp.at[pl.ds(sid * _RPT * 8, _RPT * 8)])
            plsc.subcore_barrier()

            for t in range(_ET):
                for bb in range(h):
                    pltpu.sync_copy(
                        atT_hbm.at[pl.ds((t * 2 * h + bb) * 10240, _N)],
                        ai_v.at[pl.ds(bb * 10240, _N)])
                    pltpu.sync_copy(
                        atT_hbm.at[pl.ds((t * 2 * h + h + bb) * 10240, _N)],
                        aj_v.at[pl.ds(bb * 10240, _N)])

                base_t = t * _EPT + sid * _TPS

                def chunk(c0, carry):
                    base = base_t + c0 * _CH
                    pltpu.sync_copy(src_hbm.at[pl.ds(base, _CH)], srcb)
                    pltpu.sync_copy(dst_hbm.at[pl.ds(base, _CH)], dstb)
                    for g in range(_CH // 16):
                        dv = dstb[pl.ds(g * 16, 16)]
                        sv = srcb[pl.ds(g * 16, 16)]
                        sadj[pl.ds(g * 16, 16)] = (
                            sv + (cid * npass + p) * _N)
                        for bb in range(h):
                            ai = plsc.load_gather(ai_v, [dv + bb * 10240])
                            aj = plsc.load_gather(aj_v, [sv + bb * 10240])
                            uu = plsc.load_gather(ub_v, [dv + bb * 10240])
                            s = ai + aj
                            s = jnp.where(s >= 0, s, 0.2 * s)
                            ex = jnp.exp(s - uu)
                            exb[pl.ds(bb * _CH + g * 16, 16)] = ex
                            dind[pl.ds(bb * _CH + g * 16, 16)] = dv * 8 + bb

                    if p == 0:
                        @pl.when(cid == 0)
                        def _():
                            pltpu.sync_copy(exb, den_sp.at[dind], add=True)

                    pltpu.async_copy(lin_hbm.at[sadj], rows_v, sem).wait()

                    bw = cid * _CH if h == 2 else 0

                    def wbody(i, wc):
                        exs = plsc.load_gather(
                            exb, [jnp.full((16,), bw + i, jnp.int32)])
                        for j in range(acc_w // 16):
                            rows_v[i, pl.ds(j * 16, 16)] = (
                                rows_v[i, pl.ds(j * 16, 16)] * exs)
                        return wc
                    lax.fori_loop(0, _CH, wbody, 0)

                    pltpu.sync_copy(rows_v, acc_sp.at[dstb], add=True)
                    return carry

                lax.fori_loop(0, _NCH, chunk, 0)

            plsc.subcore_barrier()

            qbase = (cid * npass + p) * _N
            pltpu.sync_copy(acc_sp.at[pl.ds(sid * 624, 624)],
                            agg_hbm.at[pl.ds(qbase + sid * 624, 624)])

            @pl.when(sid == 15)
            def _():
                pltpu.sync_copy(acc_sp.at[pl.ds(9984, 16)],
                                agg_hbm.at[pl.ds(qbase + 9984, 16)])

            if p == 0:
                @pl.when(cid == 0)
                def _():
                    pltpu.sync_copy(
                        den_sp.at[pl.ds(sid * _RPT * 8, _RPT * 8)],
                        den_hbm.at[pl.ds(sid * _RPT * 8, _RPT * 8)])
            if npass > 1 and p + 1 < npass:
                plsc.subcore_barrier()

    kfn = pl.kernel(
        body, mesh=mesh,
        compiler_params=pltpu.CompilerParams(needs_layout_passes=False,
                                             use_tc_tiling_on_sc=False),
        out_type=[jax.ShapeDtypeStruct((nq * _N, acc_w), jnp.float32),
                  jax.ShapeDtypeStruct((8 * _N,), jnp.float32)],
        scratch_types=[
            pltpu.VMEM_SHARED((_N, acc_w), jnp.float32),
            pltpu.VMEM_SHARED((8 * _N,), jnp.float32),
            pltpu.VMEM((h * 10240,), jnp.float32),
            pltpu.VMEM((h * 10240,), jnp.float32),
            pltpu.VMEM((h * 10240,), jnp.float32),
            pltpu.VMEM((125, acc_w), jnp.float32),
            pltpu.VMEM((((_RPT * 8 + 15) // 16) * 16,), jnp.float32),
            pltpu.VMEM((_CH,), jnp.int32),
            pltpu.VMEM((_CH,), jnp.int32),
            pltpu.VMEM((_CH,), jnp.int32),
            pltpu.VMEM((h * _CH,), jnp.int32),
            pltpu.VMEM((h * _CH,), jnp.float32),
            pltpu.VMEM((_CH, acc_w), jnp.float32),
            pltpu.SemaphoreType.DMA,
        ])
    return kfn(lin_stk, atT, ubT, src, dst)


def _final_stage(aggL, aggR, denT, bias, o2):
    def body(aggL_ref, aggR_ref, denT_ref, bias_ref, o2_ref,
             a4_ref, mean_ref, acc_ref):
        b = pl.program_id(0)
        agg = jnp.concatenate([aggL_ref[...], aggR_ref[...]], axis=1)
        d = denT_ref[...][:, :1]
        o4 = agg / (d + 1e-16) + bias_ref[0]
        a4 = jnp.maximum(o4 + o2_ref[...], 0.0)
        a4_ref[...] = a4

        @pl.when(b == 0)
        def _():
            acc_ref[...] = jnp.zeros((8, 64), jnp.float32)

        s = jnp.sum(a4, axis=0)
        row = jnp.where(b < 4, 0, jnp.where(b < 6, 1,
                                            jnp.where(b < 8, 2, 3)))
        upd = jnp.zeros((8, 64), jnp.float32)
        sel = (lax.broadcasted_iota(jnp.int32, (8, 64), 0) == row)
        upd = jnp.where(sel, s[None, :], 0.0)

        @pl.when(b < 8)
        def _():
            acc_ref[...] = acc_ref[...] + upd

        ri = lax.broadcasted_iota(jnp.int32, (8, 64), 0)
        cnt = jnp.where(ri == 0, 4000.0, jnp.where(ri < 3, 2000.0, 1.0))
        mean_ref[...] = acc_ref[...] / cnt

    return pl.pallas_call(
        body, grid=(_NB,),
        in_specs=[pl.BlockSpec((_BN, 32), lambda b: (b, 0)),
                  pl.BlockSpec((_BN, 32), lambda b: (b, 0)),
                  pl.BlockSpec((_BN, 8), lambda b: (b, 0)),
                  pl.BlockSpec((1, 1, 64), lambda b: (_blk_type(b), 0, 0)),
                  pl.BlockSpec((_BN, 64), lambda b: (b, 0))],
        out_specs=[pl.BlockSpec((_BN, 64), lambda b: (b, 0)),
                   pl.BlockSpec((8, 64), lambda b: (0, 0))],
        out_shape=[jax.ShapeDtypeStruct((_N, 64), jnp.float32),
                   jax.ShapeDtypeStruct((8, 64), jnp.float32)],
        scratch_shapes=[pltpu.VMEM((8, 64), jnp.float32)])(
            aggL, aggR, denT, bias.reshape(_NT, 1, 64), o2)


def _head(choices, act, Wl, bl, Wr, br, W1, b1, W2, b2):
    rows, K, lc = choices.shape
    rp = max(rows, 8)
    bn = 200 if rows >= 500 else rp
    cp = jnp.pad(choices.reshape(rows * K, lc),
                 ((0, (rp - rows) * K), (0, 128 - lc)))
    ap = jnp.pad(act, ((0, rp - rows), (0, 0)))
    Wlp = jnp.pad(Wl, ((0, 128 - lc), (0, 0)))
    W2p = jnp.pad(W2, ((0, 0), (0, 127)))
    b2p = jnp.pad(b2, (0, 127))

    def body(cp_ref, a_ref, wl_ref, bl_ref, wr_ref, br_ref, w1_ref, b1_ref,
             w2_ref, b2_ref, out_ref):
        left = jnp.dot(cp_ref[...], wl_ref[...],
                       preferred_element_type=jnp.float32) + bl_ref[...]
        right = jnp.dot(a_ref[...], wr_ref[...],
                        preferred_element_type=jnp.float32) + br_ref[...]
        rightb = jnp.broadcast_to(right[:, None, :],
                                  (bn, K, 128)).reshape(bn * K, 128)
        t = jnp.maximum(jnp.concatenate([left, rightb], axis=1), 0.0)
        t = jnp.maximum(jnp.dot(t, w1_ref[...],
                                preferred_element_type=jnp.float32)
                        + b1_ref[...], 0.0)
        lt = jnp.dot(t, w2_ref[...],
                     preferred_element_type=jnp.float32) + b2_ref[...]
        l0 = lt[:, :1].reshape(bn, K)
        m = jnp.max(l0, axis=1, keepdims=True)
        ex = jnp.exp(l0 - m)
        out_ref[...] = ex / jnp.sum(ex, axis=1, keepdims=True)

    out = pl.pallas_call(
        body, grid=(rp // bn,),
        in_specs=[pl.BlockSpec((bn * K, 128), lambda b: (b, 0)),
                  pl.BlockSpec((bn, 64), lambda b: (b, 0)),
                  pl.BlockSpec((128, 128), lambda b: (0, 0)),
                  pl.BlockSpec((1, 128), lambda b: (0, 0)),
                  pl.BlockSpec((64, 128), lambda b: (0, 0)),
                  pl.BlockSpec((1, 128), lambda b: (0, 0)),
                  pl.BlockSpec((256, 256), lambda b: (0, 0)),
                  pl.BlockSpec((1, 256), lambda b: (0, 0)),
                  pl.BlockSpec((256, 128), lambda b: (0, 0)),
                  pl.BlockSpec((1, 128), lambda b: (0, 0))],
        out_specs=pl.BlockSpec((bn, K), lambda b: (b, 0)),
        out_shape=jax.ShapeDtypeStruct((rp, K), jnp.float32))(
            cp, ap, Wlp, bl.reshape(1, 128), Wr, br.reshape(1, 128),
            W1, b1.reshape(1, 256), W2p, b2p.reshape(1, 128))
    return out[:rows, :, None]


def _wat(edge_w, h, oc):
    wl = edge_w[:, :, :oc]
    wr = edge_w[:, :, oc:]
    stacked = jnp.stack([wl, wr], axis=2)
    eye = jnp.eye(h, dtype=jnp.float32)
    tmp = jnp.einsum('ab,tbsc->actsb', eye, stacked)
    w = tmp.reshape(h * oc, _ET * 2 * h)
    return jnp.pad(w, ((0, 0), (0, 32 - _ET * 2 * h)))


def kernel(x, edge_index, node_type_index, edge_type_index, l1_node_w, l1_edge_w, l1_node_b, l2_node_w, l2_edge_w, l2_node_b, l3_node_w, l3_edge_w, l3_node_b, l4_node_w, l4_edge_w, l4_node_b, spatial_Wl, spatial_bl, spatial_Wr, spatial_br, spatial_W1, spatial_b1, spatial_W2, spatial_b2, reduce_Wl, reduce_bl, reduce_Wr, reduce_br, reduce_W1, reduce_b1, reduce_W2, reduce_b2, fuse_Wl, fuse_bl, fuse_Wr, fuse_br, fuse_W1, fuse_b1, fuse_W2, fuse_b2, reorder_Wl, reorder_bl, reorder_Wr, reorder_br, reorder_W1, reorder_b1, reorder_W2, reorder_b2, unroll_Wl, unroll_bl, unroll_Wr, unroll_br, unroll_W1, unroll_b1, unroll_W2, unroll_b2, spatial_choices, reduce_choices, fuse_choices, reorder_choices, unroll_choices):
    src = edge_index[0]
    dst = edge_index[1]

    cfgs = [(l1_node_w, l1_edge_w, l1_node_b, 1, 64),
            (l2_node_w, l2_edge_w, l2_node_b, 1, 64),
            (l3_node_w, l3_edge_w, l3_node_b, 2, 128),
            (l4_node_w, l4_edge_w, l4_node_b, 1, 64)]

    prev = None
    o2 = None
    den = None
    for li, (nw, ew, nb, h, oc) in enumerate(cfgs):
        f = h * oc
        half = f // 2
        acc_w = half if h == 1 else 64
        nq = f // acc_w
        wat = _wat(ew, h, oc)
        if li == 0:
            lin_q, at = _dense_stage(x, nw, wat, f, half, nq, acc_w)
        else:
            hp, ocp = cfgs[li - 1][3], cfgs[li - 1][4]
            prev_t = (prev, den, cfgs[li - 1][2], hp, ocp)
            outs = _dense_stage(None, nw, wat, f, half, nq, acc_w,
                                prev=prev_t, emit_o=(li == 2))
            if li == 2:
                lin_q, at, o2 = outs
            else:
                lin_q, at = outs
        atT = _transpose_stage(jnp.pad(at, ((0, 240), (0, 0))))
        ubT = _ub_stage(atT, h)
        lin_stk = lin_q.reshape(nq * _N, acc_w)
        agg, den_flat = _sc_edge_stage(lin_stk, atT.reshape(-1),
                                       ubT.reshape(-1), src, dst, half, h)
        den = den_flat.reshape(_N, 8)
        agg2 = agg.reshape(nq, _N, acc_w)
        prev = [agg2[q] for q in range(nq)]

    a4, means = _final_stage(prev[0], prev[1], den, l4_node_b, o2)

    tensor_mean = means[0]
    spatial_mean = means[1]
    reduce_mean = means[2]
    spatial_act = a4[4000:6000]
    reduce_act = a4[6000:8000]

    spatial_out = _head(spatial_choices, spatial_act, spatial_Wl, spatial_bl,
                        spatial_Wr, spatial_br, spatial_W1, spatial_b1,
                        spatial_W2, spatial_b2)
    reduce_out = _head(reduce_choices, reduce_act, reduce_Wl, reduce_bl,
                       reduce_Wr, reduce_br, reduce_W1, reduce_b1,
                       reduce_W2, reduce_b2)
    fuse_act = jnp.broadcast_to(spatial_mean[None, :], (4, 64))
    fuse_out = _head(fuse_choices, fuse_act, fuse_Wl, fuse_bl, fuse_Wr,
                     fuse_br, fuse_W1, fuse_b1, fuse_W2, fuse_b2)
    reorder_act = jnp.broadcast_to(reduce_mean[None, :], (4, 64))
    reorder_out = _head(reorder_choices, reorder_act, reorder_Wl, reorder_bl,
                        reorder_Wr, reorder_br, reorder_W1, reorder_b1,
                        reorder_W2, reorder_b2)
    unroll_act = jnp.broadcast_to(tensor_mean[None, :], (4, 64))
    unroll_out = _head(unroll_choices, unroll_act, unroll_Wl, unroll_bl,
                       unroll_Wr, unroll_br, unroll_W1, unroll_b1,
                       unroll_W2, unroll_b2)
    return (spatial_out, reduce_out, fuse_out, reorder_out, unroll_out)


# speedup vs baseline: 17.9525x; 1.1928x over previous
import functools

import jax
import jax.numpy as jnp
from jax import lax
from jax.experimental import pallas as pl
from jax.experimental.pallas import tpu as pltpu
from jax.experimental.pallas import tpu_sc as plsc

_N = 10000
_E = 320000
_NT = 4
_ET = 5
_NTI = [0, 4000, 6000, 8000, 10000]
_BN = 1000
_NB = _N // _BN
_CH = 80
_EPT = _E // _ET
_TPS = _EPT // 16
_NCH = _TPS // _CH
_RPT = _N // 16


def _blk_type(b):
    b = jnp.asarray(b)
    return ((b >= 4).astype(jnp.int32) + (b >= 6).astype(jnp.int32)
            + (b >= 8).astype(jnp.int32))


def _dense_stage(a_prev, nw, wat, f, half, nq, acc_w, prev=None,
                 emit_o=False):
    ic = nw.shape[1]
    npz = 0 if prev is None else len(prev[0])

    def body(*refs):
        if prev is None:
            a_ref, nw_ref, wat_ref = refs[:3]
            outs = refs[3:]
            a = a_ref[...]
        else:
            part_refs = refs[:npz]
            denT_ref, bias_ref, nw_ref, wat_ref = refs[npz:npz + 4]
            outs = refs[npz + 4:]
            hp, ocp = prev[3], prev[4]
            agg = jnp.concatenate([r[...] for r in part_refs], axis=1)
            d = denT_ref[...][:, :hp]
            o = agg.reshape(_BN, hp, ocp) / (d[:, :, None] + 1e-16)
            o = o.reshape(_BN, hp * ocp) + bias_ref[0]
            if emit_o:
                outs[2][...] = o
            a = jnp.maximum(o, 0.0)
        lin = jnp.maximum(jnp.dot(a, nw_ref[0],
                                  preferred_element_type=jnp.float32), 0.0)
        at = jnp.dot(lin, wat_ref[...], preferred_element_type=jnp.float32)
        for qq in range(nq):
            outs[0][qq] = lin[:, qq * acc_w:(qq + 1) * acc_w]
        outs[1][...] = at

    out_shapes = [jax.ShapeDtypeStruct((nq, _N, acc_w), jnp.float32),
                  jax.ShapeDtypeStruct((_N, 32), jnp.float32)]
    out_specs = [pl.BlockSpec((nq, _BN, acc_w), lambda b: (0, b, 0)),
                 pl.BlockSpec((_BN, 32), lambda b: (b, 0))]
    if prev is None:
        in_specs = [pl.BlockSpec((_BN, ic), lambda b: (b, 0)),
                    pl.BlockSpec((1, ic, f), lambda b: (_blk_type(b), 0, 0)),
                    pl.BlockSpec((f, 32), lambda b: (0, 0))]
        args = (a_prev, nw, wat)
    else:
        parts, den, bias, hp, ocp = prev
        fp = hp * ocp
        if emit_o:
            out_shapes.append(jax.ShapeDtypeStruct((_N, fp), jnp.float32))
            out_specs.append(pl.BlockSpec((_BN, fp), lambda b: (b, 0)))
        aw = parts[0].shape[1]
        in_specs = ([pl.BlockSpec((_BN, aw), lambda b: (b, 0))
                     for _ in range(npz)]
                    + [pl.BlockSpec((_BN, 8), lambda b: (b, 0)),
                       pl.BlockSpec((1, 1, fp),
                                    lambda b: (_blk_type(b), 0, 0)),
                       pl.BlockSpec((1, ic, f),
                                    lambda b: (_blk_type(b), 0, 0)),
                       pl.BlockSpec((f, 32), lambda b: (0, 0))])
        args = tuple(parts) + (den, bias.reshape(_NT, 1, fp), nw, wat)

    return pl.pallas_call(
        body, grid=(_NB,), in_specs=in_specs, out_specs=out_specs,
        out_shape=out_shapes)(*args)


def _transpose_stage(atp):
    def body(a_ref, o_ref):
        o_ref[...] = a_ref[...].T

    return pl.pallas_call(
        body, grid=(8,),
        in_specs=[pl.BlockSpec((1280, 32), lambda b: (b, 0))],
        out_specs=pl.BlockSpec((32, 1280), lambda b: (0, b)),
        out_shape=jax.ShapeDtypeStruct((32, 10240), jnp.float32))(atp)


def _ub_stage(atT, h):
    def body(at_ref, out_ref, mx_ref):
        p = pl.program_id(0)
        b = pl.program_id(1)

        @pl.when(jnp.logical_and(p == 0, b == 0))
        def _():
            mx_ref[...] = jnp.full((32, 1280), -jnp.inf, jnp.float32)

        @pl.when(p == 0)
        def _():
            mx_ref[...] = jnp.maximum(mx_ref[...], at_ref[...])

        maj = jnp.max(mx_ref[...], axis=1)
        blk = at_ref[...]
        rows = []
        for bb in range(h):
            u = jnp.full((1280,), -jnp.inf, jnp.float32)
            for t in range(_ET):
                u = jnp.maximum(u, blk[t * 2 * h + bb, :]
                                + maj[t * 2 * h + h + bb])
            rows.append(jnp.where(u >= 0, u, 0.2 * u))
        for bb in range(8 - h):
            rows.append(jnp.zeros((1280,), jnp.float32))
        out_ref[...] = jnp.stack(rows, axis=0)

    return pl.pallas_call(
        body, grid=(2, 8),
        in_specs=[pl.BlockSpec((32, 1280), lambda p, b: (0, b))],
        out_specs=pl.BlockSpec((8, 1280), lambda p, b: (0, b)),
        out_shape=jax.ShapeDtypeStruct((8, 10240), jnp.float32),
        scratch_shapes=[pltpu.VMEM((32, 1280), jnp.float32)])(atT)


def _sc_edge_stage(lin_stk, atT, ubT, src, dst, half, h):
    acc_w = 32
    npass = half // acc_w
    nq = 2 * npass
    mesh = plsc.VectorSubcoreMesh(core_axis_name="c", subcore_axis_name="s",
                                  num_cores=2)

    def body(lin_hbm, atT_hbm, ubT_hbm, src_hbm, dst_hbm,
             agg_hbm, den_hbm,
             acc_sp, den_sp, ai_v, aj_v, ub_v, zb_v, zd_v,
             srcT, dstT, dstb, sadj, dind, exb, rows_v, sem):
        cid = lax.axis_index("c")
        sid = lax.axis_index("s")

        def zb_body(i, zc):
            for j in range(acc_w // 16):
                zb_v[i, pl.ds(j * 16, 16)] = jnp.zeros((16,), jnp.float32)
            return zc
        lax.fori_loop(0, 125, zb_body, 0)

        def zd_body(i, zc):
            zd_v[pl.ds(i * 16, 16)] = jnp.zeros((16,), jnp.float32)
            return zc
        lax.fori_loop(0, (_RPT * 8 + 15) // 16, zd_body, 0)

        for bb in range(h):
            pltpu.sync_copy(ubT_hbm.at[pl.ds(bb * 10240, _N)],
                            ub_v.at[pl.ds(bb * 10240, _N)])

        for p in range(npass):
            for k in range(5):
                pltpu.sync_copy(zb_v,
                                acc_sp.at[pl.ds(sid * _RPT + k * 125, 125)])
            if p == 0:
                pltpu.sync_copy(zd_v.at[pl.ds(0, _RPT * 8)],
                                den_sp.at[pl.ds(sid * _RPT * 8, _RPT * 8)])
            plsc.subcore_barrier()

            for t in range(_ET):
                for bb in range(h):
                    pltpu.sync_copy(
                        atT_hbm.at[pl.ds((t * 2 * h + bb) * 10240, _N)],
                        ai_v.at[pl.ds(bb * 10240, _N)])
                    pltpu.sync_copy(
                        atT_hbm.at[pl.ds((t * 2 * h + h + bb) * 10240, _N)],
                        aj_v.at[pl.ds(bb * 10240, _N)])

                base_t = t * _EPT + sid * _TPS
                pltpu.sync_copy(src_hbm.at[pl.ds(base_t, _TPS)], srcT)
                pltpu.sync_copy(dst_hbm.at[pl.ds(base_t, _TPS)], dstT)

                def chunk(c0, carry):
                    e0 = c0 * _CH
                    def ibody(g, ic_):
                        sadj[pl.ds(g * 16, 16)] = (
                            srcT[pl.ds(e0 + g * 16, 16)]
                            + (cid * npass + p) * _N)
                        dstb[pl.ds(g * 16, 16)] = dstT[pl.ds(e0 + g * 16,
                                                             16)]
                        return ic_
                    lax.fori_loop(0, _CH // 16, ibody, 0)
                    cp = pltpu.async_copy(lin_hbm.at[sadj], rows_v, sem)

                    for g in range(_CH // 16):
                        dv = dstb[pl.ds(g * 16, 16)]
                        sv = srcT[pl.ds(e0 + g * 16, 16)]
                        for bb in range(h):
                            ai = plsc.load_gather(ai_v, [dv + bb * 10240])
                            aj = plsc.load_gather(aj_v, [sv + bb * 10240])
                            uu = plsc.load_gather(ub_v, [dv + bb * 10240])
                            s = ai + aj
                            s = jnp.where(s >= 0, s, 0.2 * s)
                            ex = jnp.exp(s - uu)
                            exb[pl.ds(bb * _CH + g * 16, 16)] = ex
                            dind[pl.ds(bb * _CH + g * 16, 16)] = dv * 8 + bb

                    if p == 0:
                        @pl.when(cid == 0)
                        def _():
                            pltpu.sync_copy(exb, den_sp.at[dind], add=True)

                    cp.wait()

                    bw = cid * _CH if h == 2 else 0

                    def wbody(i, wc):
                        exs = plsc.load_gather(
                            exb, [jnp.full((16,), bw + i, jnp.int32)])
                        for j in range(acc_w // 16):
                            rows_v[i, pl.ds(j * 16, 16)] = (
                                rows_v[i, pl.ds(j * 16, 16)] * exs)
                        return wc
                    lax.fori_loop(0, _CH, wbody, 0)

                    pltpu.sync_copy(rows_v, acc_sp.at[dstb], add=True)
                    return carry

                lax.fori_loop(0, _NCH, chunk, 0)

            plsc.subcore_barrier()

            qbase = (cid * npass + p) * _N
            pltpu.sync_copy(acc_sp.at[pl.ds(sid * 624, 624)],
                            agg_hbm.at[pl.ds(qbase + sid * 624, 624)])

            @pl.when(sid == 15)
            def _():
                pltpu.sync_copy(acc_sp.at[pl.ds(9984, 16)],
                                agg_hbm.at[pl.ds(qbase + 9984, 16)])

            if p == 0:
                @pl.when(cid == 0)
                def _():
                    pltpu.sync_copy(
                        den_sp.at[pl.ds(sid * _RPT * 8, _RPT * 8)],
                        den_hbm.at[pl.ds(sid * _RPT * 8, _RPT * 8)])
            if npass > 1 and p + 1 < npass:
                plsc.subcore_barrier()

    kfn = pl.kernel(
        body, mesh=mesh,
        compiler_params=pltpu.CompilerParams(needs_layout_passes=False,
                                             use_tc_tiling_on_sc=False),
        out_type=[jax.ShapeDtypeStruct((nq * _N, acc_w), jnp.float32),
                  jax.ShapeDtypeStruct((8 * _N,), jnp.float32)],
        scratch_types=[
            pltpu.VMEM_SHARED((_N, acc_w), jnp.float32),
            pltpu.VMEM_SHARED((8 * _N,), jnp.float32),
            pltpu.VMEM((h * 10240,), jnp.float32),
            pltpu.VMEM((h * 10240,), jnp.float32),
            pltpu.VMEM((h * 10240,), jnp.float32),
            pltpu.VMEM((125, acc_w), jnp.float32),
            pltpu.VMEM((((_RPT * 8 + 15) // 16) * 16,), jnp.float32),
            pltpu.VMEM((_TPS,), jnp.int32),
            pltpu.VMEM((_TPS,), jnp.int32),
            pltpu.VMEM((_CH,), jnp.int32),
            pltpu.VMEM((_CH,), jnp.int32),
            pltpu.VMEM((h * _CH,), jnp.int32),
            pltpu.VMEM((h * _CH,), jnp.float32),
            pltpu.VMEM((_CH, acc_w), jnp.float32),
            pltpu.SemaphoreType.DMA,
        ])
    return kfn(lin_stk, atT, ubT, src, dst)


def _final_stage(aggL, aggR, denT, bias, o2):
    def body(aggL_ref, aggR_ref, denT_ref, bias_ref, o2_ref,
             a4_ref, mean_ref, acc_ref):
        b = pl.program_id(0)
        agg = jnp.concatenate([aggL_ref[...], aggR_ref[...]], axis=1)
        d = denT_ref[...][:, :1]
        o4 = agg / (d + 1e-16) + bias_ref[0]
        a4 = jnp.maximum(o4 + o2_ref[...], 0.0)
        a4_ref[...] = a4

        @pl.when(b == 0)
        def _():
            acc_ref[...] = jnp.zeros((8, 64), jnp.float32)

        s = jnp.sum(a4, axis=0)
        row = jnp.where(b < 4, 0, jnp.where(b < 6, 1,
                                            jnp.where(b < 8, 2, 3)))
        upd = jnp.zeros((8, 64), jnp.float32)
        sel = (lax.broadcasted_iota(jnp.int32, (8, 64), 0) == row)
        upd = jnp.where(sel, s[None, :], 0.0)

        @pl.when(b < 8)
        def _():
            acc_ref[...] = acc_ref[...] + upd

        ri = lax.broadcasted_iota(jnp.int32, (8, 64), 0)
        cnt = jnp.where(ri == 0, 4000.0, jnp.where(ri < 3, 2000.0, 1.0))
        mean_ref[...] = acc_ref[...] / cnt

    return pl.pallas_call(
        body, grid=(_NB,),
        in_specs=[pl.BlockSpec((_BN, 32), lambda b: (b, 0)),
                  pl.BlockSpec((_BN, 32), lambda b: (b, 0)),
                  pl.BlockSpec((_BN, 8), lambda b: (b, 0)),
                  pl.BlockSpec((1, 1, 64), lambda b: (_blk_type(b), 0, 0)),
                  pl.BlockSpec((_BN, 64), lambda b: (b, 0))],
        out_specs=[pl.BlockSpec((_BN, 64), lambda b: (b, 0)),
                   pl.BlockSpec((8, 64), lambda b: (0, 0))],
        out_shape=[jax.ShapeDtypeStruct((_N, 64), jnp.float32),
                   jax.ShapeDtypeStruct((8, 64), jnp.float32)],
        scratch_shapes=[pltpu.VMEM((8, 64), jnp.float32)])(
            aggL, aggR, denT, bias.reshape(_NT, 1, 64), o2)


def _head(choices, act, Wl, bl, Wr, br, W1, b1, W2, b2):
    rows, K, lc = choices.shape
    rp = max(rows, 8)
    bn = 200 if rows >= 500 else rp
    cp = jnp.pad(choices.reshape(rows * K, lc),
                 ((0, (rp - rows) * K), (0, 128 - lc)))
    ap = jnp.pad(act, ((0, rp - rows), (0, 0)))
    Wlp = jnp.pad(Wl, ((0, 128 - lc), (0, 0)))
    W2p = jnp.pad(W2, ((0, 0), (0, 127)))
    b2p = jnp.pad(b2, (0, 127))

    def body(cp_ref, a_ref, wl_ref, bl_ref, wr_ref, br_ref, w1_ref, b1_ref,
             w2_ref, b2_ref, out_ref):
        left = jnp.dot(cp_ref[...], wl_ref[...],
                       preferred_element_type=jnp.float32) + bl_ref[...]
        right = jnp.dot(a_ref[...], wr_ref[...],
                        preferred_element_type=jnp.float32) + br_ref[...]
        rightb = jnp.broadcast_to(right[:, None, :],
                                  (bn, K, 128)).reshape(bn * K, 128)
        t = jnp.maximum(jnp.concatenate([left, rightb], axis=1), 0.0)
        t = jnp.maximum(jnp.dot(t, w1_ref[...],
                                preferred_element_type=jnp.float32)
                        + b1_ref[...], 0.0)
        lt = jnp.dot(t, w2_ref[...],
                     preferred_element_type=jnp.float32) + b2_ref[...]
        l0 = lt[:, :1].reshape(bn, K)
        m = jnp.max(l0, axis=1, keepdims=True)
        ex = jnp.exp(l0 - m)
        out_ref[...] = ex / jnp.sum(ex, axis=1, keepdims=True)

    out = pl.pallas_call(
        body, grid=(rp // bn,),
        in_specs=[pl.BlockSpec((bn * K, 128), lambda b: (b, 0)),
                  pl.BlockSpec((bn, 64), lambda b: (b, 0)),
                  pl.BlockSpec((128, 128), lambda b: (0, 0)),
                  pl.BlockSpec((1, 128), lambda b: (0, 0)),
                  pl.BlockSpec((64, 128), lambda b: (0, 0)),
                  pl.BlockSpec((1, 128), lambda b: (0, 0)),
                  pl.BlockSpec((256, 256), lambda b: (0, 0)),
                  pl.BlockSpec((1, 256), lambda b: (0, 0)),
                  pl.BlockSpec((256, 128), lambda b: (0, 0)),
                  pl.BlockSpec((1, 128), lambda b: (0, 0))],
        out_specs=pl.BlockSpec((bn, K), lambda b: (b, 0)),
        out_shape=jax.ShapeDtypeStruct((rp, K), jnp.float32))(
            cp, ap, Wlp, bl.reshape(1, 128), Wr, br.reshape(1, 128),
            W1, b1.reshape(1, 256), W2p, b2p.reshape(1, 128))
    return out[:rows, :, None]


def _wat(edge_w, h, oc):
    wl = edge_w[:, :, :oc]
    wr = edge_w[:, :, oc:]
    stacked = jnp.stack([wl, wr], axis=2)
    eye = jnp.eye(h, dtype=jnp.float32)
    tmp = jnp.einsum('ab,tbsc->actsb', eye, stacked)
    w = tmp.reshape(h * oc, _ET * 2 * h)
    return jnp.pad(w, ((0, 0), (0, 32 - _ET * 2 * h)))


def kernel(x, edge_index, node_type_index, edge_type_index, l1_node_w, l1_edge_w, l1_node_b, l2_node_w, l2_edge_w, l2_node_b, l3_node_w, l3_edge_w, l3_node_b, l4_node_w, l4_edge_w, l4_node_b, spatial_Wl, spatial_bl, spatial_Wr, spatial_br, spatial_W1, spatial_b1, spatial_W2, spatial_b2, reduce_Wl, reduce_bl, reduce_Wr, reduce_br, reduce_W1, reduce_b1, reduce_W2, reduce_b2, fuse_Wl, fuse_bl, fuse_Wr, fuse_br, fuse_W1, fuse_b1, fuse_W2, fuse_b2, reorder_Wl, reorder_bl, reorder_Wr, reorder_br, reorder_W1, reorder_b1, reorder_W2, reorder_b2, unroll_Wl, unroll_bl, unroll_Wr, unroll_br, unroll_W1, unroll_b1, unroll_W2, unroll_b2, spatial_choices, reduce_choices, fuse_choices, reorder_choices, unroll_choices):
    src = edge_index[0]
    dst = edge_index[1]

    cfgs = [(l1_node_w, l1_edge_w, l1_node_b, 1, 64),
            (l2_node_w, l2_edge_w, l2_node_b, 1, 64),
            (l3_node_w, l3_edge_w, l3_node_b, 2, 128),
            (l4_node_w, l4_edge_w, l4_node_b, 1, 64)]

    prev = None
    o2 = None
    den = None
    for li, (nw, ew, nb, h, oc) in enumerate(cfgs):
        f = h * oc
        half = f // 2
        acc_w = 32
        nq = f // acc_w
        wat = _wat(ew, h, oc)
        if li == 0:
            lin_q, at = _dense_stage(x, nw, wat, f, half, nq, acc_w)
        else:
            hp, ocp = cfgs[li - 1][3], cfgs[li - 1][4]
            prev_t = (prev, den, cfgs[li - 1][2], hp, ocp)
            outs = _dense_stage(None, nw, wat, f, half, nq, acc_w,
                                prev=prev_t, emit_o=(li == 2))
            if li == 2:
                lin_q, at, o2 = outs
            else:
                lin_q, at = outs
        atT = _transpose_stage(jnp.pad(at, ((0, 240), (0, 0))))
        ubT = _ub_stage(atT, h)
        lin_stk = lin_q.reshape(nq * _N, acc_w)
        agg, den_flat = _sc_edge_stage(lin_stk, atT.reshape(-1),
                                       ubT.reshape(-1), src, dst, half, h)
        den = den_flat.reshape(_N, 8)
        agg2 = agg.reshape(nq, _N, acc_w)
        prev = [agg2[q] for q in range(nq)]

    a4, means = _final_stage(prev[0], prev[1], den, l4_node_b, o2)

    tensor_mean = means[0]
    spatial_mean = means[1]
    reduce_mean = means[2]
    spatial_act = a4[4000:6000]
    reduce_act = a4[6000:8000]

    spatial_out = _head(spatial_choices, spatial_act, spatial_Wl, spatial_bl,
                        spatial_Wr, spatial_br, spatial_W1, spatial_b1,
                        spatial_W2, spatial_b2)
    reduce_out = _head(reduce_choices, reduce_act, reduce_Wl, reduce_bl,
                       reduce_Wr, reduce_br, reduce_W1, reduce_b1,
                       reduce_W2, reduce_b2)
    fuse_act = jnp.broadcast_to(spatial_mean[None, :], (4, 64))
    fuse_out = _head(fuse_choices, fuse_act, fuse_Wl, fuse_bl, fuse_Wr,
                     fuse_br, fuse_W1, fuse_b1, fuse_W2, fuse_b2)
    reorder_act = jnp.broadcast_to(reduce_mean[None, :], (4, 64))
    reorder_out = _head(reorder_choices, reorder_act, reorder_Wl, reorder_bl,
                        reorder_Wr, reorder_br, reorder_W1, reorder_b1,
                        reorder_W2, reorder_b2)
    unroll_act = jnp.broadcast_to(tensor_mean[None, :], (4, 64))
    unroll_out = _head(unroll_choices, unroll_act, unroll_Wl, unroll_bl,
                       unroll_Wr, unroll_br, unroll_W1, unroll_b1,
                       unroll_W2, unroll_b2)
    return (spatial_out, reduce_out, fuse_out, reorder_out, unroll_out)


# speedup vs baseline: 22.0201x; 1.2266x over previous
import functools

import jax
import jax.numpy as jnp
from jax import lax
from jax.experimental import pallas as pl
from jax.experimental.pallas import tpu as pltpu
from jax.experimental.pallas import tpu_sc as plsc

_N = 10000
_E = 320000
_NT = 4
_ET = 5
_NTI = [0, 4000, 6000, 8000, 10000]
_BN = 1000
_NB = _N // _BN
_CH = 80
_EPT = _E // _ET
_TPS = _EPT // 16
_NCH = _TPS // _CH
_RPT = _N // 16


def _blk_type(b):
    b = jnp.asarray(b)
    return ((b >= 4).astype(jnp.int32) + (b >= 6).astype(jnp.int32)
            + (b >= 8).astype(jnp.int32))


def _dense_stage(a_prev, nw, wat, f, half, nq, acc_w, prev=None,
                 emit_o=False):
    ic = nw.shape[1]
    npz = 0 if prev is None else len(prev[0])

    def body(*refs):
        if prev is None:
            a_ref, nw_ref, wat_ref = refs[:3]
            outs = refs[3:]
            a = a_ref[...]
        else:
            part_refs = refs[:npz]
            denT_ref, bias_ref, nw_ref, wat_ref = refs[npz:npz + 4]
            outs = refs[npz + 4:]
            hp, ocp = prev[3], prev[4]
            agg = jnp.concatenate([r[...] for r in part_refs], axis=1)
            d = denT_ref[...][:, :hp]
            o = agg.reshape(_BN, hp, ocp) / (d[:, :, None] + 1e-16)
            o = o.reshape(_BN, hp * ocp) + bias_ref[0]
            if emit_o:
                outs[2][...] = o
            a = jnp.maximum(o, 0.0)
        lin = jnp.maximum(jnp.dot(a, nw_ref[0],
                                  preferred_element_type=jnp.float32), 0.0)
        at = jnp.dot(lin, wat_ref[...], preferred_element_type=jnp.float32)
        for qq in range(nq):
            outs[0][qq] = lin[:, qq * acc_w:(qq + 1) * acc_w]
        outs[1][...] = at

    out_shapes = [jax.ShapeDtypeStruct((nq, _N, acc_w), jnp.float32),
                  jax.ShapeDtypeStruct((_N, 32), jnp.float32)]
    out_specs = [pl.BlockSpec((nq, _BN, acc_w), lambda b: (0, b, 0)),
                 pl.BlockSpec((_BN, 32), lambda b: (b, 0))]
    if prev is None:
        in_specs = [pl.BlockSpec((_BN, ic), lambda b: (b, 0)),
                    pl.BlockSpec((1, ic, f), lambda b: (_blk_type(b), 0, 0)),
                    pl.BlockSpec((f, 32), lambda b: (0, 0))]
        args = (a_prev, nw, wat)
    else:
        parts, den, bias, hp, ocp = prev
        fp = hp * ocp
        if emit_o:
            out_shapes.append(jax.ShapeDtypeStruct((_N, fp), jnp.float32))
            out_specs.append(pl.BlockSpec((_BN, fp), lambda b: (b, 0)))
        aw = parts[0].shape[1]
        in_specs = ([pl.BlockSpec((_BN, aw), lambda b: (b, 0))
                     for _ in range(npz)]
                    + [pl.BlockSpec((_BN, 2), lambda b: (b, 0)),
                       pl.BlockSpec((1, 1, fp),
                                    lambda b: (_blk_type(b), 0, 0)),
                       pl.BlockSpec((1, ic, f),
                                    lambda b: (_blk_type(b), 0, 0)),
                       pl.BlockSpec((f, 32), lambda b: (0, 0))])
        args = tuple(parts) + (den, bias.reshape(_NT, 1, fp), nw, wat)

    return pl.pallas_call(
        body, grid=(_NB,), in_specs=in_specs, out_specs=out_specs,
        out_shape=out_shapes)(*args)


def _transpose_stage(atp):
    def body(a_ref, o_ref):
        o_ref[...] = a_ref[...].T

    return pl.pallas_call(
        body, grid=(8,),
        in_specs=[pl.BlockSpec((1280, 32), lambda b: (b, 0))],
        out_specs=pl.BlockSpec((32, 1280), lambda b: (0, b)),
        out_shape=jax.ShapeDtypeStruct((32, 10240), jnp.float32))(atp)


def _ub_stage(atT, h):
    def body(at_ref, out_ref, mx_ref):
        p = pl.program_id(0)
        b = pl.program_id(1)

        @pl.when(jnp.logical_and(p == 0, b == 0))
        def _():
            mx_ref[...] = jnp.full((32, 1280), -jnp.inf, jnp.float32)

        @pl.when(p == 0)
        def _():
            mx_ref[...] = jnp.maximum(mx_ref[...], at_ref[...])

        maj = jnp.max(mx_ref[...], axis=1)
        blk = at_ref[...]
        rows = []
        for bb in range(h):
            u = jnp.full((1280,), -jnp.inf, jnp.float32)
            for t in range(_ET):
                u = jnp.maximum(u, blk[t * 2 * h + bb, :]
                                + maj[t * 2 * h + h + bb])
            rows.append(jnp.where(u >= 0, u, 0.2 * u))
        for bb in range(8 - h):
            rows.append(jnp.zeros((1280,), jnp.float32))
        out_ref[...] = jnp.stack(rows, axis=0)

    return pl.pallas_call(
        body, grid=(2, 8),
        in_specs=[pl.BlockSpec((32, 1280), lambda p, b: (0, b))],
        out_specs=pl.BlockSpec((8, 1280), lambda p, b: (0, b)),
        out_shape=jax.ShapeDtypeStruct((8, 10240), jnp.float32),
        scratch_shapes=[pltpu.VMEM((32, 1280), jnp.float32)])(atT)


def _sc_edge_stage(lin_stk, atT, ubT, src, dst, half, h):
    acc_w = half if h == 1 else 64
    npass = half // acc_w
    nq = 2 * npass
    mesh = plsc.VectorSubcoreMesh(core_axis_name="c", subcore_axis_name="s",
                                  num_cores=2)

    def body(lin_hbm, atT_hbm, ubT_hbm, src_hbm, dst_hbm,
             agg_hbm, den_hbm,
             acc_sp, den_sp, ai_v, aj_v, ub_v, zb_v, zd_v,
             srcT, dstT, dstb, sadj, dind, exb, rows_v, sem):
        cid = lax.axis_index("c")
        sid = lax.axis_index("s")

        def zb_body(i, zc):
            for j in range(acc_w // 16):
                zb_v[i, pl.ds(j * 16, 16)] = jnp.zeros((16,), jnp.float32)
            return zc
        lax.fori_loop(0, 125, zb_body, 0)

        def zd_body(i, zc):
            zd_v[pl.ds(i * 16, 16)] = jnp.zeros((16,), jnp.float32)
            return zc
        lax.fori_loop(0, 78, zd_body, 0)

        for bb in range(h):
            pltpu.sync_copy(ubT_hbm.at[pl.ds(bb * 10240, _N)],
                            ub_v.at[pl.ds(bb * 10240, _N)])

        for p in range(npass):
            for k in range(5):
                pltpu.sync_copy(zb_v,
                                acc_sp.at[pl.ds(sid * _RPT + k * 125, 125)])
            if p == 0:
                pltpu.sync_copy(zd_v.at[pl.ds(0, 1248)],
                                den_sp.at[pl.ds(sid * 1248, 1248)])

                @pl.when(sid == 15)
                def _():
                    pltpu.sync_copy(zd_v.at[pl.ds(0, 32)],
                                    den_sp.at[pl.ds(19968, 32)])
            plsc.subcore_barrier()

            for t in range(_ET):
                for bb in range(h):
                    pltpu.sync_copy(
                        atT_hbm.at[pl.ds((t * 2 * h + bb) * 10240, _N)],
                        ai_v.at[pl.ds(bb * 10240, _N)])
                    pltpu.sync_copy(
                        atT_hbm.at[pl.ds((t * 2 * h + h + bb) * 10240,
                                         _N)],
                        aj_v.at[pl.ds(bb * 10240, _N)])

                base_t = t * _EPT + sid * _TPS
                pltpu.sync_copy(src_hbm.at[pl.ds(base_t, _TPS)], srcT)
                pltpu.sync_copy(dst_hbm.at[pl.ds(base_t, _TPS)], dstT)

                def chunk(c0, carry):
                    e0 = c0 * _CH
                    def ibody(g, ic_):
                        sadj[pl.ds(g * 16, 16)] = (
                            srcT[pl.ds(e0 + g * 16, 16)]
                            + (cid * npass + p) * _N)
                        dstb[pl.ds(g * 16, 16)] = dstT[pl.ds(e0 + g * 16,
                                                             16)]
                        return ic_
                    lax.fori_loop(0, _CH // 16, ibody, 0)
                    cp = pltpu.async_copy(lin_hbm.at[sadj], rows_v, sem)

                    for g in range(_CH // 16):
                        dv = dstb[pl.ds(g * 16, 16)]
                        sv = srcT[pl.ds(e0 + g * 16, 16)]
                        for bb in range(h):
                            ai = plsc.load_gather(ai_v, [dv + bb * 10240])
                            aj = plsc.load_gather(aj_v, [sv + bb * 10240])
                            uu = plsc.load_gather(ub_v, [dv + bb * 10240])
                            s = ai + aj
                            s = jnp.where(s >= 0, s, 0.2 * s)
                            ex = jnp.exp(s - uu)
                            exb[pl.ds(bb * _CH + g * 16, 16)] = ex
                            dind[pl.ds(bb * _CH + g * 16, 16)] = dv * 2 + bb

                    if p == 0:
                        @pl.when(cid == 0)
                        def _():
                            pltpu.sync_copy(exb, den_sp.at[dind], add=True)

                    cp.wait()

                    bw = cid * _CH if h == 2 else 0

                    def wbody(i, wc):
                        exs = plsc.load_gather(
                            exb, [jnp.full((16,), bw + i, jnp.int32)])
                        for j in range(acc_w // 16):
                            rows_v[i, pl.ds(j * 16, 16)] = (
                                rows_v[i, pl.ds(j * 16, 16)] * exs)
                        return wc
                    lax.fori_loop(0, _CH, wbody, 0)

                    pltpu.sync_copy(rows_v, acc_sp.at[dstb], add=True)
                    return carry

                lax.fori_loop(0, _NCH, chunk, 0)

            plsc.subcore_barrier()

            qbase = (cid * npass + p) * _N
            pltpu.sync_copy(acc_sp.at[pl.ds(sid * 624, 624)],
                            agg_hbm.at[pl.ds(qbase + sid * 624, 624)])

            @pl.when(sid == 15)
            def _():
                pltpu.sync_copy(acc_sp.at[pl.ds(9984, 16)],
                                agg_hbm.at[pl.ds(qbase + 9984, 16)])

            if p == 0:
                @pl.when(cid == 0)
                def _():
                    pltpu.sync_copy(den_sp.at[pl.ds(sid * 1248, 1248)],
                                    den_hbm.at[pl.ds(sid * 1248, 1248)])

                @pl.when(jnp.logical_and(cid == 0, sid == 15))
                def _():
                    pltpu.sync_copy(den_sp.at[pl.ds(19968, 32)],
                                    den_hbm.at[pl.ds(19968, 32)])
            if npass > 1 and p + 1 < npass:
                plsc.subcore_barrier()

    kfn = pl.kernel(
        body, mesh=mesh,
        compiler_params=pltpu.CompilerParams(needs_layout_passes=False,
                                             use_tc_tiling_on_sc=False),
        out_type=[jax.ShapeDtypeStruct((nq * _N, acc_w), jnp.float32),
                  jax.ShapeDtypeStruct((2 * _N,), jnp.float32)],
        scratch_types=[
            pltpu.VMEM_SHARED((_N, acc_w), jnp.float32),
            pltpu.VMEM_SHARED((2 * _N,), jnp.float32),
            pltpu.VMEM((h * 10240,), jnp.float32),
            pltpu.VMEM((h * 10240,), jnp.float32),
            pltpu.VMEM((h * 10240,), jnp.float32),
            pltpu.VMEM((125, acc_w), jnp.float32),
            pltpu.VMEM((1248,), jnp.float32),
            pltpu.VMEM((_TPS,), jnp.int32),
            pltpu.VMEM((_TPS,), jnp.int32),
            pltpu.VMEM((_CH,), jnp.int32),
            pltpu.VMEM((_CH,), jnp.int32),
            pltpu.VMEM((h * _CH,), jnp.int32),
            pltpu.VMEM((h * _CH,), jnp.float32),
            pltpu.VMEM((_CH, acc_w), jnp.float32),
            pltpu.SemaphoreType.DMA,
        ])
    return kfn(lin_stk, atT, ubT, src, dst)


def _final_stage(aggL, aggR, denT, bias, o2):
    def body(aggL_ref, aggR_ref, denT_ref, bias_ref, o2_ref,
             a4_ref, mean_ref, acc_ref):
        b = pl.program_id(0)
        agg = jnp.concatenate([aggL_ref[...], aggR_ref[...]], axis=1)
        d = denT_ref[...][:, :1]
        o4 = agg / (d + 1e-16) + bias_ref[0]
        a4 = jnp.maximum(o4 + o2_ref[...], 0.0)
        a4_ref[...] = a4

        @pl.when(b == 0)
        def _():
            acc_ref[...] = jnp.zeros((8, 64), jnp.float32)

        s = jnp.sum(a4, axis=0)
        row = jnp.where(b < 4, 0, jnp.where(b < 6, 1,
                                            jnp.where(b < 8, 2, 3)))
        upd = jnp.zeros((8, 64), jnp.float32)
        sel = (lax.broadcasted_iota(jnp.int32, (8, 64), 0) == row)
        upd = jnp.where(sel, s[None, :], 0.0)

        @pl.when(b < 8)
        def _():
            acc_ref[...] = acc_ref[...] + upd

        ri = lax.broadcasted_iota(jnp.int32, (8, 64), 0)
        cnt = jnp.where(ri == 0, 4000.0, jnp.where(ri < 3, 2000.0, 1.0))
        mean_ref[...] = acc_ref[...] / cnt

    return pl.pallas_call(
        body, grid=(_NB,),
        in_specs=[pl.BlockSpec((_BN, 32), lambda b: (b, 0)),
                  pl.BlockSpec((_BN, 32), lambda b: (b, 0)),
                  pl.BlockSpec((_BN, 2), lambda b: (b, 0)),
                  pl.BlockSpec((1, 1, 64), lambda b: (_blk_type(b), 0, 0)),
                  pl.BlockSpec((_BN, 64), lambda b: (b, 0))],
        out_specs=[pl.BlockSpec((_BN, 64), lambda b: (b, 0)),
                   pl.BlockSpec((8, 64), lambda b: (0, 0))],
        out_shape=[jax.ShapeDtypeStruct((_N, 64), jnp.float32),
                   jax.ShapeDtypeStruct((8, 64), jnp.float32)],
        scratch_shapes=[pltpu.VMEM((8, 64), jnp.float32)])(
            aggL, aggR, denT, bias.reshape(_NT, 1, 64), o2)


def _head(choices, act, Wl, bl, Wr, br, W1, b1, W2, b2):
    rows, K, lc = choices.shape
    rp = max(rows, 8)
    bn = 200 if rows >= 500 else rp
    cp = jnp.pad(choices.reshape(rows * K, lc),
                 ((0, (rp - rows) * K), (0, 128 - lc)))
    ap = jnp.pad(act, ((0, rp - rows), (0, 0)))
    Wlp = jnp.pad(Wl, ((0, 128 - lc), (0, 0)))
    W2p = jnp.pad(W2, ((0, 0), (0, 127)))
    b2p = jnp.pad(b2, (0, 127))

    def body(cp_ref, a_ref, wl_ref, bl_ref, wr_ref, br_ref, w1_ref, b1_ref,
             w2_ref, b2_ref, out_ref):
        left = jnp.dot(cp_ref[...], wl_ref[...],
                       preferred_element_type=jnp.float32) + bl_ref[...]
        right = jnp.dot(a_ref[...], wr_ref[...],
                        preferred_element_type=jnp.float32) + br_ref[...]
        rightb = jnp.broadcast_to(right[:, None, :],
                                  (bn, K, 128)).reshape(bn * K, 128)
        t = jnp.maximum(jnp.concatenate([left, rightb], axis=1), 0.0)
        t = jnp.maximum(jnp.dot(t, w1_ref[...],
                                preferred_element_type=jnp.float32)
                        + b1_ref[...], 0.0)
        lt = jnp.dot(t, w2_ref[...],
                     preferred_element_type=jnp.float32) + b2_ref[...]
        l0 = lt[:, :1].reshape(bn, K)
        m = jnp.max(l0, axis=1, keepdims=True)
        ex = jnp.exp(l0 - m)
        out_ref[...] = ex / jnp.sum(ex, axis=1, keepdims=True)

    out = pl.pallas_call(
        body, grid=(rp // bn,),
        in_specs=[pl.BlockSpec((bn * K, 128), lambda b: (b, 0)),
                  pl.BlockSpec((bn, 64), lambda b: (b, 0)),
                  pl.BlockSpec((128, 128), lambda b: (0, 0)),
                  pl.BlockSpec((1, 128), lambda b: (0, 0)),
                  pl.BlockSpec((64, 128), lambda b: (0, 0)),
                  pl.BlockSpec((1, 128), lambda b: (0, 0)),
                  pl.BlockSpec((256, 256), lambda b: (0, 0)),
                  pl.BlockSpec((1, 256), lambda b: (0, 0)),
                  pl.BlockSpec((256, 128), lambda b: (0, 0)),
                  pl.BlockSpec((1, 128), lambda b: (0, 0))],
        out_specs=pl.BlockSpec((bn, K), lambda b: (b, 0)),
        out_shape=jax.ShapeDtypeStruct((rp, K), jnp.float32))(
            cp, ap, Wlp, bl.reshape(1, 128), Wr, br.reshape(1, 128),
            W1, b1.reshape(1, 256), W2p, b2p.reshape(1, 128))
    return out[:rows, :, None]


def _wat(edge_w, h, oc):
    wl = edge_w[:, :, :oc]
    wr = edge_w[:, :, oc:]
    stacked = jnp.stack([wl, wr], axis=2)
    eye = jnp.eye(h, dtype=jnp.float32)
    tmp = jnp.einsum('ab,tbsc->actsb', eye, stacked)
    w = tmp.reshape(h * oc, _ET * 2 * h)
    return jnp.pad(w, ((0, 0), (0, 32 - _ET * 2 * h)))


def kernel(x, edge_index, node_type_index, edge_type_index, l1_node_w, l1_edge_w, l1_node_b, l2_node_w, l2_edge_w, l2_node_b, l3_node_w, l3_edge_w, l3_node_b, l4_node_w, l4_edge_w, l4_node_b, spatial_Wl, spatial_bl, spatial_Wr, spatial_br, spatial_W1, spatial_b1, spatial_W2, spatial_b2, reduce_Wl, reduce_bl, reduce_Wr, reduce_br, reduce_W1, reduce_b1, reduce_W2, reduce_b2, fuse_Wl, fuse_bl, fuse_Wr, fuse_br, fuse_W1, fuse_b1, fuse_W2, fuse_b2, reorder_Wl, reorder_bl, reorder_Wr, reorder_br, reorder_W1, reorder_b1, reorder_W2, reorder_b2, unroll_Wl, unroll_bl, unroll_Wr, unroll_br, unroll_W1, unroll_b1, unroll_W2, unroll_b2, spatial_choices, reduce_choices, fuse_choices, reorder_choices, unroll_choices):
    src = edge_index[0]
    dst = edge_index[1]

    cfgs = [(l1_node_w, l1_edge_w, l1_node_b, 1, 64),
            (l2_node_w, l2_edge_w, l2_node_b, 1, 64),
            (l3_node_w, l3_edge_w, l3_node_b, 2, 128),
            (l4_node_w, l4_edge_w, l4_node_b, 1, 64)]

    prev = None
    o2 = None
    den = None
    for li, (nw, ew, nb, h, oc) in enumerate(cfgs):
        f = h * oc
        half = f // 2
        acc_w = half if h == 1 else 64
        nq = f // acc_w
        wat = _wat(ew, h, oc)
        if li == 0:
            lin_q, at = _dense_stage(x, nw, wat, f, half, nq, acc_w)
        else:
            hp, ocp = cfgs[li - 1][3], cfgs[li - 1][4]
            prev_t = (prev, den, cfgs[li - 1][2], hp, ocp)
            outs = _dense_stage(None, nw, wat, f, half, nq, acc_w,
                                prev=prev_t, emit_o=(li == 2))
            if li == 2:
                lin_q, at, o2 = outs
            else:
                lin_q, at = outs
        atT = _transpose_stage(jnp.pad(at, ((0, 240), (0, 0))))
        ubT = _ub_stage(atT, h)
        lin_stk = lin_q.reshape(nq * _N, acc_w)
        agg, den_flat = _sc_edge_stage(lin_stk, atT.reshape(-1),
                                       ubT.reshape(-1), src, dst, half, h)
        den = den_flat.reshape(_N, 2)
        agg2 = agg.reshape(nq, _N, acc_w)
        prev = [agg2[q] for q in range(nq)]

    a4, means = _final_stage(prev[0], prev[1], den, l4_node_b, o2)

    tensor_mean = means[0]
    spatial_mean = means[1]
    reduce_mean = means[2]
    spatial_act = a4[4000:6000]
    reduce_act = a4[6000:8000]

    spatial_out = _head(spatial_choices, spatial_act, spatial_Wl, spatial_bl,
                        spatial_Wr, spatial_br, spatial_W1, spatial_b1,
                        spatial_W2, spatial_b2)
    reduce_out = _head(reduce_choices, reduce_act, reduce_Wl, reduce_bl,
                       reduce_Wr, reduce_br, reduce_W1, reduce_b1,
                       reduce_W2, reduce_b2)
    fuse_act = jnp.broadcast_to(spatial_mean[None, :], (4, 64))
    fuse_out = _head(fuse_choices, fuse_act, fuse_Wl, fuse_bl, fuse_Wr,
                     fuse_br, fuse_W1, fuse_b1, fuse_W2, fuse_b2)
    reorder_act = jnp.broadcast_to(reduce_mean[None, :], (4, 64))
    reorder_out = _head(reorder_choices, reorder_act, reorder_Wl, reorder_bl,
                        reorder_Wr, reorder_br, reorder_W1, reorder_b1,
                        reorder_W2, reorder_b2)
    unroll_act = jnp.broadcast_to(tensor_mean[None, :], (4, 64))
    unroll_out = _head(unroll_choices, unroll_act, unroll_Wl, unroll_bl,
                       unroll_Wr, unroll_br, unroll_W1, unroll_b1,
                       unroll_W2, unroll_b2)
    return (spatial_out, reduce_out, fuse_out, reorder_out, unroll_out)


# speedup vs baseline: 29.0489x; 1.3192x over previous
import functools

import jax
import jax.numpy as jnp
from jax import lax
from jax.experimental import pallas as pl
from jax.experimental.pallas import tpu as pltpu
from jax.experimental.pallas import tpu_sc as plsc

_N = 10000
_E = 320000
_NT = 4
_ET = 5
_NTI = [0, 4000, 6000, 8000, 10000]
_BN = 1000
_NB = _N // _BN
_CH = 80
_EPT = _E // _ET
_TPS = _EPT // 16
_NCH = _TPS // _CH
_RPT = _N // 16


def _blk_type(b):
    b = jnp.asarray(b)
    return ((b >= 4).astype(jnp.int32) + (b >= 6).astype(jnp.int32)
            + (b >= 8).astype(jnp.int32))


def _dense_stage(a_prev, nw, wat, f, half, nq, acc_w, prev=None,
                 emit_o=False):
    ic = nw.shape[1]
    npz = 0 if prev is None else len(prev[0])

    def body(*refs):
        if prev is None:
            a_ref, nw_ref, wat_ref = refs[:3]
            outs = refs[3:]
            a = a_ref[...]
        else:
            part_refs = refs[:npz]
            denT_ref, bias_ref, nw_ref, wat_ref = refs[npz:npz + 4]
            outs = refs[npz + 4:]
            hp, ocp = prev[3], prev[4]
            agg = jnp.concatenate([r[...] for r in part_refs], axis=1)
            d = denT_ref[...][:, :hp]
            o = agg.reshape(_BN, hp, ocp) / (d[:, :, None] + 1e-16)
            o = o.reshape(_BN, hp * ocp) + bias_ref[0]
            if emit_o:
                outs[2][...] = o
            a = jnp.maximum(o, 0.0)
        lin = jnp.maximum(jnp.dot(a, nw_ref[0],
                                  preferred_element_type=jnp.float32), 0.0)
        at = jnp.dot(lin, wat_ref[...], preferred_element_type=jnp.float32)
        for qq in range(nq):
            outs[0][qq] = lin[:, qq * acc_w:(qq + 1) * acc_w]
        outs[1][...] = at

    out_shapes = [jax.ShapeDtypeStruct((nq, _N, acc_w), jnp.float32),
                  jax.ShapeDtypeStruct((_N, 32), jnp.float32)]
    out_specs = [pl.BlockSpec((nq, _BN, acc_w), lambda b: (0, b, 0)),
                 pl.BlockSpec((_BN, 32), lambda b: (b, 0))]
    if prev is None:
        in_specs = [pl.BlockSpec((_BN, ic), lambda b: (b, 0)),
                    pl.BlockSpec((1, ic, f), lambda b: (_blk_type(b), 0, 0)),
                    pl.BlockSpec((f, 32), lambda b: (0, 0))]
        args = (a_prev, nw, wat)
    else:
        parts, den, bias, hp, ocp = prev
        fp = hp * ocp
        if emit_o:
            out_shapes.append(jax.ShapeDtypeStruct((_N, fp), jnp.float32))
            out_specs.append(pl.BlockSpec((_BN, fp), lambda b: (b, 0)))
        aw = parts[0].shape[1]
        in_specs = ([pl.BlockSpec((_BN, aw), lambda b: (b, 0))
                     for _ in range(npz)]
                    + [pl.BlockSpec((_BN, 2), lambda b: (b, 0)),
                       pl.BlockSpec((1, 1, fp),
                                    lambda b: (_blk_type(b), 0, 0)),
                       pl.BlockSpec((1, ic, f),
                                    lambda b: (_blk_type(b), 0, 0)),
                       pl.BlockSpec((f, 32), lambda b: (0, 0))])
        args = tuple(parts) + (den, bias.reshape(_NT, 1, fp), nw, wat)

    return pl.pallas_call(
        body, grid=(_NB,), in_specs=in_specs, out_specs=out_specs,
        out_shape=out_shapes)(*args)


def _transpose_stage(atp):
    def body(a_ref, o_ref):
        o_ref[...] = a_ref[...].T

    return pl.pallas_call(
        body, grid=(8,),
        in_specs=[pl.BlockSpec((1280, 32), lambda b: (b, 0))],
        out_specs=pl.BlockSpec((32, 1280), lambda b: (0, b)),
        out_shape=jax.ShapeDtypeStruct((32, 10240), jnp.float32))(atp)


def _ub_stage(atT, h):
    def body(at_ref, out_ref, mx_ref):
        p = pl.program_id(0)
        b = pl.program_id(1)

        @pl.when(jnp.logical_and(p == 0, b == 0))
        def _():
            mx_ref[...] = jnp.full((32, 1280), -jnp.inf, jnp.float32)

        @pl.when(p == 0)
        def _():
            mx_ref[...] = jnp.maximum(mx_ref[...], at_ref[...])

        maj = jnp.max(mx_ref[...], axis=1)
        blk = at_ref[...]
        rows = []
        for bb in range(h):
            u = jnp.full((1280,), -jnp.inf, jnp.float32)
            for t in range(_ET):
                u = jnp.maximum(u, blk[t * 2 * h + bb, :]
                                + maj[t * 2 * h + h + bb])
            rows.append(jnp.where(u >= 0, u, 0.2 * u))
        for bb in range(8 - h):
            rows.append(jnp.zeros((1280,), jnp.float32))
        out_ref[...] = jnp.stack(rows, axis=0)

    return pl.pallas_call(
        body, grid=(2, 8),
        in_specs=[pl.BlockSpec((32, 1280), lambda p, b: (0, b))],
        out_specs=pl.BlockSpec((8, 1280), lambda p, b: (0, b)),
        out_shape=jax.ShapeDtypeStruct((8, 10240), jnp.float32),
        scratch_shapes=[pltpu.VMEM((32, 1280), jnp.float32)])(atT)


def _sc_edge_stage(lin_stk, atT, ubT, src, dst, half, h):
    acc_w = half if h == 1 else 64
    npass = half // acc_w
    nq = 2 * npass
    mesh = plsc.VectorSubcoreMesh(core_axis_name="c", subcore_axis_name="s",
                                  num_cores=2)

    def body(lin_hbm, atT_hbm, ubT_hbm, src_hbm, dst_hbm,
             agg_hbm, den_hbm,
             acc_sp, den_sp, ai_v, aj_v, ub_v, zb_v, zd_v,
             srcT, dstT, dstb, sadj, dind, exb, rows_v,
             dstb2, sadj2, rows_v2, sem, sem2):
        cid = lax.axis_index("c")
        sid = lax.axis_index("s")

        def zb_body(i, zc):
            for j in range(acc_w // 16):
                zb_v[i, pl.ds(j * 16, 16)] = jnp.zeros((16,), jnp.float32)
            return zc
        lax.fori_loop(0, 125, zb_body, 0)

        def zd_body(i, zc):
            zd_v[pl.ds(i * 16, 16)] = jnp.zeros((16,), jnp.float32)
            return zc
        lax.fori_loop(0, 78, zd_body, 0)

        for bb in range(h):
            pltpu.sync_copy(ubT_hbm.at[pl.ds(bb * 10240, _N)],
                            ub_v.at[pl.ds(bb * 10240, _N)])

        for p in range(npass):
            for k in range(5):
                pltpu.sync_copy(zb_v,
                                acc_sp.at[pl.ds(sid * _RPT + k * 125, 125)])
            if p == 0:
                pltpu.sync_copy(zd_v.at[pl.ds(0, 1248)],
                                den_sp.at[pl.ds(sid * 1248, 1248)])

                @pl.when(sid == 15)
                def _():
                    pltpu.sync_copy(zd_v.at[pl.ds(0, 32)],
                                    den_sp.at[pl.ds(19968, 32)])
            plsc.subcore_barrier()

            for t in range(_ET):
                for bb in range(h):
                    pltpu.sync_copy(
                        atT_hbm.at[pl.ds((t * 2 * h + bb) * 10240, _N)],
                        ai_v.at[pl.ds(bb * 10240, _N)])
                    pltpu.sync_copy(
                        atT_hbm.at[pl.ds((t * 2 * h + h + bb) * 10240,
                                         _N)],
                        aj_v.at[pl.ds(bb * 10240, _N)])

                base_t = t * _EPT + sid * _TPS
                pltpu.sync_copy(src_hbm.at[pl.ds(base_t, _TPS)], srcT)
                pltpu.sync_copy(dst_hbm.at[pl.ds(base_t, _TPS)], dstT)

                qoff = (cid * npass + p) * _N

                def prep(c0, sadj_b, dstb_b):
                    e0 = c0 * _CH

                    def ibody(g, ic_):
                        sadj_b[pl.ds(g * 16, 16)] = (
                            srcT[pl.ds(e0 + g * 16, 16)] + qoff)
                        dstb_b[pl.ds(g * 16, 16)] = dstT[pl.ds(e0 + g * 16,
                                                               16)]
                        return ic_
                    lax.fori_loop(0, _CH // 16, ibody, 0)

                def exwork(c0, dstb_b):
                    e0 = c0 * _CH
                    for g in range(_CH // 16):
                        dv = dstb_b[pl.ds(g * 16, 16)]
                        sv = srcT[pl.ds(e0 + g * 16, 16)]
                        for bb in range(h):
                            ai = plsc.load_gather(ai_v, [dv + bb * 10240])
                            aj = plsc.load_gather(aj_v, [sv + bb * 10240])
                            uu = plsc.load_gather(ub_v, [dv + bb * 10240])
                            s = ai + aj
                            s = jnp.where(s >= 0, s, 0.2 * s)
                            ex = jnp.exp(s - uu)
                            exb[pl.ds(bb * _CH + g * 16, 16)] = ex
                            dind[pl.ds(bb * _CH + g * 16, 16)] = dv * 2 + bb
                    if p == 0:
                        @pl.when(cid == 0)
                        def _():
                            pltpu.sync_copy(exb, den_sp.at[dind], add=True)

                def drain(rows_b, dstb_b, sem_b):
                    pltpu.make_async_copy(lin_hbm.at[dstb_b], rows_b,
                                          sem_b).wait()
                    bw = cid * _CH if h == 2 else 0

                    def wbody(i, wc):
                        exs = plsc.load_gather(
                            exb, [jnp.full((16,), bw + i, jnp.int32)])
                        for j in range(acc_w // 16):
                            rows_b[i, pl.ds(j * 16, 16)] = (
                                rows_b[i, pl.ds(j * 16, 16)] * exs)
                        return wc
                    lax.fori_loop(0, _CH, wbody, 0)

                    pltpu.sync_copy(rows_b, acc_sp.at[dstb_b], add=True)

                prep(0, sadj, dstb)
                pltpu.async_copy(lin_hbm.at[sadj], rows_v, sem)

                def cpair(cc, carry):
                    c0 = 2 * cc
                    prep(c0 + 1, sadj2, dstb2)
                    pltpu.async_copy(lin_hbm.at[sadj2], rows_v2, sem2)
                    exwork(c0, dstb)
                    drain(rows_v, dstb, sem)

                    @pl.when(cc < _NCH // 2 - 1)
                    def _():
                        prep(c0 + 2, sadj, dstb)
                        pltpu.async_copy(lin_hbm.at[sadj], rows_v, sem)
                    exwork(c0 + 1, dstb2)
                    drain(rows_v2, dstb2, sem2)
                    return carry

                lax.fori_loop(0, _NCH // 2, cpair, 0)

            plsc.subcore_barrier()

            qbase = (cid * npass + p) * _N
            pltpu.sync_copy(acc_sp.at[pl.ds(sid * 624, 624)],
                            agg_hbm.at[pl.ds(qbase + sid * 624, 624)])

            @pl.when(sid == 15)
            def _():
                pltpu.sync_copy(acc_sp.at[pl.ds(9984, 16)],
                                agg_hbm.at[pl.ds(qbase + 9984, 16)])

            if p == 0:
                @pl.when(cid == 0)
                def _():
                    pltpu.sync_copy(den_sp.at[pl.ds(sid * 1248, 1248)],
                                    den_hbm.at[pl.ds(sid * 1248, 1248)])

                @pl.when(jnp.logical_and(cid == 0, sid == 15))
                def _():
                    pltpu.sync_copy(den_sp.at[pl.ds(19968, 32)],
                                    den_hbm.at[pl.ds(19968, 32)])
            if npass > 1 and p + 1 < npass:
                plsc.subcore_barrier()

    kfn = pl.kernel(
        body, mesh=mesh,
        compiler_params=pltpu.CompilerParams(needs_layout_passes=False,
                                             use_tc_tiling_on_sc=False),
        out_type=[jax.ShapeDtypeStruct((nq * _N, acc_w), jnp.float32),
                  jax.ShapeDtypeStruct((2 * _N,), jnp.float32)],
        scratch_types=[
            pltpu.VMEM_SHARED((_N, acc_w), jnp.float32),
            pltpu.VMEM_SHARED((2 * _N,), jnp.float32),
            pltpu.VMEM((h * 10240,), jnp.float32),
            pltpu.VMEM((h * 10240,), jnp.float32),
            pltpu.VMEM((h * 10240,), jnp.float32),
            pltpu.VMEM((125, acc_w), jnp.float32),
            pltpu.VMEM((1248,), jnp.float32),
            pltpu.VMEM((_TPS,), jnp.int32),
            pltpu.VMEM((_TPS,), jnp.int32),
            pltpu.VMEM((_CH,), jnp.int32),
            pltpu.VMEM((_CH,), jnp.int32),
            pltpu.VMEM((h * _CH,), jnp.int32),
            pltpu.VMEM((h * _CH,), jnp.float32),
            pltpu.VMEM((_CH, acc_w), jnp.float32),
            pltpu.VMEM((_CH,), jnp.int32),
            pltpu.VMEM((_CH,), jnp.int32),
            pltpu.VMEM((_CH, acc_w), jnp.float32),
            pltpu.SemaphoreType.DMA,
            pltpu.SemaphoreType.DMA,
        ])
    return kfn(lin_stk, atT, ubT, src, dst)


def _final_stage(aggL, aggR, denT, bias, o2):
    def body(aggL_ref, aggR_ref, denT_ref, bias_ref, o2_ref,
             a4_ref, mean_ref, acc_ref):
        b = pl.program_id(0)
        agg = jnp.concatenate([aggL_ref[...], aggR_ref[...]], axis=1)
        d = denT_ref[...][:, :1]
        o4 = agg / (d + 1e-16) + bias_ref[0]
        a4 = jnp.maximum(o4 + o2_ref[...], 0.0)
        a4_ref[...] = a4

        @pl.when(b == 0)
        def _():
            acc_ref[...] = jnp.zeros((8, 64), jnp.float32)

        s = jnp.sum(a4, axis=0)
        row = jnp.where(b < 4, 0, jnp.where(b < 6, 1,
                                            jnp.where(b < 8, 2, 3)))
        upd = jnp.zeros((8, 64), jnp.float32)
        sel = (lax.broadcasted_iota(jnp.int32, (8, 64), 0) == row)
        upd = jnp.where(sel, s[None, :], 0.0)

        @pl.when(b < 8)
        def _():
            acc_ref[...] = acc_ref[...] + upd

        ri = lax.broadcasted_iota(jnp.int32, (8, 64), 0)
        cnt = jnp.where(ri == 0, 4000.0, jnp.where(ri < 3, 2000.0, 1.0))
        mean_ref[...] = acc_ref[...] / cnt

    return pl.pallas_call(
        body, grid=(_NB,),
        in_specs=[pl.BlockSpec((_BN, 32), lambda b: (b, 0)),
                  pl.BlockSpec((_BN, 32), lambda b: (b, 0)),
                  pl.BlockSpec((_BN, 2), lambda b: (b, 0)),
                  pl.BlockSpec((1, 1, 64), lambda b: (_blk_type(b), 0, 0)),
                  pl.BlockSpec((_BN, 64), lambda b: (b, 0))],
        out_specs=[pl.BlockSpec((_BN, 64), lambda b: (b, 0)),
                   pl.BlockSpec((8, 64), lambda b: (0, 0))],
        out_shape=[jax.ShapeDtypeStruct((_N, 64), jnp.float32),
                   jax.ShapeDtypeStruct((8, 64), jnp.float32)],
        scratch_shapes=[pltpu.VMEM((8, 64), jnp.float32)])(
            aggL, aggR, denT, bias.reshape(_NT, 1, 64), o2)


def _head(choices, act, Wl, bl, Wr, br, W1, b1, W2, b2):
    rows, K, lc = choices.shape
    rp = max(rows, 8)
    bn = 200 if rows >= 500 else rp
    cp = jnp.pad(choices.reshape(rows * K, lc),
                 ((0, (rp - rows) * K), (0, 128 - lc)))
    ap = jnp.pad(act, ((0, rp - rows), (0, 0)))
    Wlp = jnp.pad(Wl, ((0, 128 - lc), (0, 0)))
    W2p = jnp.pad(W2, ((0, 0), (0, 127)))
    b2p = jnp.pad(b2, (0, 127))

    def body(cp_ref, a_ref, wl_ref, bl_ref, wr_ref, br_ref, w1_ref, b1_ref,
             w2_ref, b2_ref, out_ref):
        left = jnp.dot(cp_ref[...], wl_ref[...],
                       preferred_element_type=jnp.float32) + bl_ref[...]
        right = jnp.dot(a_ref[...], wr_ref[...],
                        preferred_element_type=jnp.float32) + br_ref[...]
        rightb = jnp.broadcast_to(right[:, None, :],
                                  (bn, K, 128)).reshape(bn * K, 128)
        t = jnp.maximum(jnp.concatenate([left, rightb], axis=1), 0.0)
        t = jnp.maximum(jnp.dot(t, w1_ref[...],
                                preferred_element_type=jnp.float32)
                        + b1_ref[...], 0.0)
        lt = jnp.dot(t, w2_ref[...],
                     preferred_element_type=jnp.float32) + b2_ref[...]
        l0 = lt[:, :1].reshape(bn, K)
        m = jnp.max(l0, axis=1, keepdims=True)
        ex = jnp.exp(l0 - m)
        out_ref[...] = ex / jnp.sum(ex, axis=1, keepdims=True)

    out = pl.pallas_call(
        body, grid=(rp // bn,),
        in_specs=[pl.BlockSpec((bn * K, 128), lambda b: (b, 0)),
                  pl.BlockSpec((bn, 64), lambda b: (b, 0)),
                  pl.BlockSpec((128, 128), lambda b: (0, 0)),
                  pl.BlockSpec((1, 128), lambda b: (0, 0)),
                  pl.BlockSpec((64, 128), lambda b: (0, 0)),
                  pl.BlockSpec((1, 128), lambda b: (0, 0)),
                  pl.BlockSpec((256, 256), lambda b: (0, 0)),
                  pl.BlockSpec((1, 256), lambda b: (0, 0)),
                  pl.BlockSpec((256, 128), lambda b: (0, 0)),
                  pl.BlockSpec((1, 128), lambda b: (0, 0))],
        out_specs=pl.BlockSpec((bn, K), lambda b: (b, 0)),
        out_shape=jax.ShapeDtypeStruct((rp, K), jnp.float32))(
            cp, ap, Wlp, bl.reshape(1, 128), Wr, br.reshape(1, 128),
            W1, b1.reshape(1, 256), W2p, b2p.reshape(1, 128))
    return out[:rows, :, None]


def _wat(edge_w, h, oc):
    wl = edge_w[:, :, :oc]
    wr = edge_w[:, :, oc:]
    stacked = jnp.stack([wl, wr], axis=2)
    eye = jnp.eye(h, dtype=jnp.float32)
    tmp = jnp.einsum('ab,tbsc->actsb', eye, stacked)
    w = tmp.reshape(h * oc, _ET * 2 * h)
    return jnp.pad(w, ((0, 0), (0, 32 - _ET * 2 * h)))


def kernel(x, edge_index, node_type_index, edge_type_index, l1_node_w, l1_edge_w, l1_node_b, l2_node_w, l2_edge_w, l2_node_b, l3_node_w, l3_edge_w, l3_node_b, l4_node_w, l4_edge_w, l4_node_b, spatial_Wl, spatial_bl, spatial_Wr, spatial_br, spatial_W1, spatial_b1, spatial_W2, spatial_b2, reduce_Wl, reduce_bl, reduce_Wr, reduce_br, reduce_W1, reduce_b1, reduce_W2, reduce_b2, fuse_Wl, fuse_bl, fuse_Wr, fuse_br, fuse_W1, fuse_b1, fuse_W2, fuse_b2, reorder_Wl, reorder_bl, reorder_Wr, reorder_br, reorder_W1, reorder_b1, reorder_W2, reorder_b2, unroll_Wl, unroll_bl, unroll_Wr, unroll_br, unroll_W1, unroll_b1, unroll_W2, unroll_b2, spatial_choices, reduce_choices, fuse_choices, reorder_choices, unroll_choices):
    src = edge_index[0]
    dst = edge_index[1]

    cfgs = [(l1_node_w, l1_edge_w, l1_node_b, 1, 64),
            (l2_node_w, l2_edge_w, l2_node_b, 1, 64),
            (l3_node_w, l3_edge_w, l3_node_b, 2, 128),
            (l4_node_w, l4_edge_w, l4_node_b, 1, 64)]

    prev = None
    o2 = None
    den = None
    for li, (nw, ew, nb, h, oc) in enumerate(cfgs):
        f = h * oc
        half = f // 2
        acc_w = half if h == 1 else 64
        nq = f // acc_w
        wat = _wat(ew, h, oc)
        if li == 0:
            lin_q, at = _dense_stage(x, nw, wat, f, half, nq, acc_w)
        else:
            hp, ocp = cfgs[li - 1][3], cfgs[li - 1][4]
            prev_t = (prev, den, cfgs[li - 1][2], hp, ocp)
            outs = _dense_stage(None, nw, wat, f, half, nq, acc_w,
                                prev=prev_t, emit_o=(li == 2))
            if li == 2:
                lin_q, at, o2 = outs
            else:
                lin_q, at = outs
        atT = _transpose_stage(jnp.pad(at, ((0, 240), (0, 0))))
        ubT = _ub_stage(atT, h)
        lin_stk = lin_q.reshape(nq * _N, acc_w)
        agg, den_flat = _sc_edge_stage(lin_stk, atT.reshape(-1),
                                       ubT.reshape(-1), src, dst, half, h)
        den = den_flat.reshape(_N, 2)
        agg2 = agg.reshape(nq, _N, acc_w)
        prev = [agg2[q] for q in range(nq)]

    a4, means = _final_stage(prev[0], prev[1], den, l4_node_b, o2)

    tensor_mean = means[0]
    spatial_mean = means[1]
    reduce_mean = means[2]
    spatial_act = a4[4000:6000]
    reduce_act = a4[6000:8000]

    spatial_out = _head(spatial_choices, spatial_act, spatial_Wl, spatial_bl,
                        spatial_Wr, spatial_br, spatial_W1, spatial_b1,
                        spatial_W2, spatial_b2)
    reduce_out = _head(reduce_choices, reduce_act, reduce_Wl, reduce_bl,
                       reduce_Wr, reduce_br, reduce_W1, reduce_b1,
                       reduce_W2, reduce_b2)
    fuse_act = jnp.broadcast_to(spatial_mean[None, :], (4, 64))
    fuse_out = _head(fuse_choices, fuse_act, fuse_Wl, fuse_bl, fuse_Wr,
                     fuse_br, fuse_W1, fuse_b1, fuse_W2, fuse_b2)
    reorder_act = jnp.broadcast_to(reduce_mean[None, :], (4, 64))
    reorder_out = _head(reorder_choices, reorder_act, reorder_Wl, reorder_bl,
                        reorder_Wr, reorder_br, reorder_W1, reorder_b1,
                        reorder_W2, reorder_b2)
    unroll_act = jnp.broadcast_to(tensor_mean[None, :], (4, 64))
    unroll_out = _head(unroll_choices, unroll_act, unroll_Wl, unroll_bl,
                       unroll_Wr, unroll_br, unroll_W1, unroll_b1,
                       unroll_W2, unroll_b2)
    return (spatial_out, reduce_out, fuse_out, reorder_out, unroll_out)
